# baseline (speedup 1.0000x reference)
import functools

import jax
import jax.numpy as jnp
from jax import lax
from jax.experimental import pallas as pl
from jax.experimental.pallas import tpu as pltpu
from jax.experimental.pallas import tpu_sc as plsc

N = 10000
E = 320000
D = 128

NC = 2
NS = 16
NW = NC * NS
LANES = 16

CHUNK = 128
CPW = 84
CHUNKS_PAD = CPW * NW
E_PAD = CHUNKS_PAD * CHUNK
RPT = 8 * (-(-N // (8 * NS)))
N_PAD = RPT * NS

_mesh = plsc.VectorSubcoreMesh(
    core_axis_name="c", subcore_axis_name="s", num_cores=NC, num_subcores=NS)


@functools.partial(
    pl.kernel,
    out_type=jax.ShapeDtypeStruct((NC, N_PAD, D), jnp.float32),
    mesh=_mesh,
    scratch_types=[
        pltpu.VMEM((2, CHUNK, D), jnp.float32),
        pltpu.VMEM((3, 2, CHUNK), jnp.int32),
        pltpu.VMEM((3, LANES, CHUNK), jnp.float32),
        pltpu.VMEM_SHARED((N_PAD, D), jnp.float32),
        pltpu.SemaphoreType.DMA, pltpu.SemaphoreType.DMA,
        pltpu.SemaphoreType.DMA, pltpu.SemaphoreType.DMA,
        pltpu.SemaphoreType.DMA,
        pltpu.SemaphoreType.DMA, pltpu.SemaphoreType.DMA,
    ],
)
def _spmm_sc(h_hbm, zeros_hbm, idx_hbm, w_hbm, out_hbm,
             rows_v, idx_v, w_v, acc_sh, sg0, sg1, si0, si1, si2, ss0, ss1):
    c = lax.axis_index("c")
    s = lax.axis_index("s")
    sg = (sg0, sg1)
    si = (si0, si1, si2)
    ss = (ss0, ss1)

    pltpu.sync_copy(zeros_hbm.at[pl.ds(s * RPT, RPT)],
                    acc_sh.at[pl.ds(s * RPT, RPT)])
    plsc.subcore_barrier()

    base = (c * NS + s) * CPW

    def meta_copy(m, j):
        pltpu.async_copy(idx_hbm.at[base + j], idx_v.at[m], si[m])
        pltpu.async_copy(w_hbm.at[base + j], w_v.at[m], si[m])

    def meta_wait(m, j):
        pltpu.make_async_copy(idx_hbm.at[base + j], idx_v.at[m],
                              si[m]).wait()
        pltpu.make_async_copy(w_hbm.at[base + j], w_v.at[m], si[m]).wait()

    def gather_start(b, m):
        pltpu.async_copy(h_hbm.at[idx_v.at[m, 0]], rows_v.at[b], sg[b])

    def gather_wait(b, m):
        pltpu.make_async_copy(h_hbm.at[idx_v.at[m, 0]], rows_v.at[b],
                              sg[b]).wait()

    meta_copy(0, 0)
    meta_wait(0, 0)
    meta_copy(1, 1)
    gather_start(0, 0)

    def scatter_start(b, m):
        pltpu.async_copy(rows_v.at[b], acc_sh.at[idx_v.at[m, 1]], ss[b],
                         add=True)

    def scatter_wait(b, m):
        pltpu.make_async_copy(rows_v.at[b], acc_sh.at[idx_v.at[m, 1]],
                              ss[b]).wait()

    def step(j, t, pf_retire, pf_gather, pf_meta):
        b = t % 2
        m = t % 3
        if pf_retire:
            scatter_wait(1 - b, (t + 2) % 3)
        if pf_gather:
            meta_wait((t + 1) % 3, j + 1)
            gather_start(1 - b, (t + 1) % 3)
        if pf_meta:
            meta_copy((t + 2) % 3, j + 2)

        gather_wait(b, m)

        def edge_body(i, carry):
            wv = w_v[m, i // 8, pl.ds((i % 8) * LANES, LANES)]
            for jj in range(D // LANES):
                sl = (b, i, pl.ds(jj * LANES, LANES))
                rows_v[sl] = rows_v[sl] * wv
            return carry
        lax.fori_loop(0, CHUNK, edge_body, 0, unroll=8)

        scatter_start(b, m)

    for t in range(6):
        step(t, t, t >= 1, True, True)

    def loop_body(jj, carry):
        for t in range(6):
            step(6 * jj + t, t, True, True, True)
        return carry
    lax.fori_loop(1, CPW // 6 - 1, loop_body, 0)
    for t in range(6):
        j = CPW - 6 + t
        step(j, t, True, j + 1 < CPW, j + 2 < CPW)

    scatter_wait(1, (CPW - 1) % 3)
    plsc.subcore_barrier()
    pltpu.sync_copy(acc_sh.at[pl.ds(s * RPT, RPT)],
                    out_hbm.at[c, pl.ds(s * RPT, RPT)])


_BN = 1000
_BC = RPT


def _combine_tc(p):
    def body(p_ref, o_ref):
        o_ref[...] = p_ref[0] + p_ref[1]
    return pl.pallas_call(
        body,
        grid=(N_PAD // _BC,),
        in_specs=[pl.BlockSpec((2, _BC, D), lambda i: (0, i, 0))],
        out_specs=pl.BlockSpec((_BC, D), lambda i: (i, 0)),
        out_shape=jax.ShapeDtypeStruct((N_PAD, D), jnp.float32),
    )(p)


def _linear_tc(h, W, b2):
    def body(h_ref, w_ref, b_ref, o_ref):
        acc = lax.dot_general(h_ref[...], w_ref[...],
                              (((1,), (1,)), ((), ())),
                              preferred_element_type=jnp.float32)
        o_ref[...] = acc + b_ref[...]
    return pl.pallas_call(
        body,
        grid=(N // _BN,),
        in_specs=[
            pl.BlockSpec((_BN, D), lambda i: (i, 0)),
            pl.BlockSpec((D, D), lambda i: (0, 0)),
            pl.BlockSpec((1, D), lambda i: (0, 0)),
        ],
        out_specs=pl.BlockSpec((_BN, D), lambda i: (i, 0)),
        out_shape=jax.ShapeDtypeStruct((N, D), jnp.float32),
    )(h, W, b2)


def kernel(x, edge_index, edge_weight, W, b, k):
    row = edge_index[0]
    col = edge_index[1]
    pad = E_PAD - E
    spread = (jnp.arange(pad, dtype=jnp.int32) * 37) % N
    col2 = jnp.concatenate([col, spread]).reshape(CHUNKS_PAD, 1, CHUNK)
    row2 = jnp.concatenate([row, spread]).reshape(CHUNKS_PAD, 1, CHUNK)
    idx = jnp.concatenate([col2, row2], axis=1)
    w2 = jnp.broadcast_to(
        jnp.pad(edge_weight, (0, pad)).reshape(CHUNKS_PAD, CHUNK, 1),
        (CHUNKS_PAD, CHUNK, LANES)).reshape(CHUNKS_PAD, LANES, CHUNK)
    zeros = jnp.zeros((N_PAD, D), jnp.float32)
    b2 = b.reshape(1, D)
    x_pad = jnp.pad(x, ((0, N_PAD - N), (0, 0)))

    def it_body(_, h):
        p = _spmm_sc(h, zeros, idx, w2)
        return _combine_tc(p)

    h = lax.fori_loop(0, k, it_body, x_pad)
    return _linear_tc(h[:N], W, b2)


# speedup vs baseline: 1.1999x; 1.1999x over previous
import functools

import jax
import jax.numpy as jnp
from jax import lax
from jax.experimental import pallas as pl
from jax.experimental.pallas import tpu as pltpu
from jax.experimental.pallas import tpu_sc as plsc

N = 10000
E = 320000
D = 128

NC = 2
NS = 16
NW = NC * NS
LANES = 16

CHUNK = 128
CPW = 80
CHUNKS_PAD = CPW * NW
E_PAD = CHUNKS_PAD * CHUNK
RPT = 8 * (-(-N // (8 * NS)))
N_PAD = RPT * NS

_mesh = plsc.VectorSubcoreMesh(
    core_axis_name="c", subcore_axis_name="s", num_cores=NC, num_subcores=NS)


@functools.partial(
    pl.kernel,
    out_type=jax.ShapeDtypeStruct((NC, N_PAD, D), jnp.float32),
    mesh=_mesh,
    scratch_types=[
        pltpu.VMEM((2, CHUNK, D), jnp.float32),
        pltpu.VMEM((4, 2, CHUNK), jnp.int32),
        pltpu.VMEM((4, LANES, CHUNK), jnp.float32),
        pltpu.VMEM_SHARED((N_PAD, D), jnp.float32),
        pltpu.SemaphoreType.DMA, pltpu.SemaphoreType.DMA,
        pltpu.SemaphoreType.DMA, pltpu.SemaphoreType.DMA,
        pltpu.SemaphoreType.DMA, pltpu.SemaphoreType.DMA,
        pltpu.SemaphoreType.DMA, pltpu.SemaphoreType.DMA,
    ],
)
def _spmm_sc(h_hbm, zeros_hbm, idx_hbm, w_hbm, out_hbm,
             rows_v, idx_v, w_v, acc_sh,
             sg0, sg1, si0, si1, si2, si3, ss0, ss1):
    c = lax.axis_index("c")
    s = lax.axis_index("s")
    sg = (sg0, sg1)
    si = (si0, si1, si2, si3)
    ss = (ss0, ss1)

    pltpu.sync_copy(zeros_hbm.at[pl.ds(s * RPT, RPT)],
                    acc_sh.at[pl.ds(s * RPT, RPT)])
    plsc.subcore_barrier()

    base = (c * NS + s) * CPW

    def meta_copy(m, j):
        pltpu.async_copy(idx_hbm.at[base + j], idx_v.at[m], si[m])
        pltpu.async_copy(w_hbm.at[base + j], w_v.at[m], si[m])

    def meta_wait(m, j):
        pltpu.make_async_copy(idx_hbm.at[base + j], idx_v.at[m],
                              si[m]).wait()
        pltpu.make_async_copy(w_hbm.at[base + j], w_v.at[m], si[m]).wait()

    def gather_start(b, m):
        pltpu.async_copy(h_hbm.at[idx_v.at[m, 0]], rows_v.at[b], sg[b])

    def gather_wait(b, m):
        pltpu.make_async_copy(h_hbm.at[idx_v.at[m, 0]], rows_v.at[b],
                              sg[b]).wait()

    meta_copy(0, 0)
    meta_wait(0, 0)
    meta_copy(1, 1)
    gather_start(0, 0)

    def scatter_start(b, m):
        pltpu.async_copy(rows_v.at[b], acc_sh.at[idx_v.at[m, 1]], ss[b],
                         add=True)

    def scatter_wait(b, m):
        pltpu.make_async_copy(rows_v.at[b], acc_sh.at[idx_v.at[m, 1]],
                              ss[b]).wait()

    def step(j, t, pf_retire, pf_gather, pf_meta):
        b = t % 2
        m = t % 4
        if pf_retire:
            scatter_wait(1 - b, (t + 3) % 4)
        if pf_gather:
            meta_wait((t + 1) % 4, j + 1)
            gather_start(1 - b, (t + 1) % 4)
        if pf_meta:
            meta_copy((t + 2) % 4, j + 2)

        gather_wait(b, m)

        def edge_body(i, carry):
            wv = w_v[m, i // 8, pl.ds((i % 8) * LANES, LANES)]
            for jj in range(D // LANES):
                sl = (b, i, pl.ds(jj * LANES, LANES))
                rows_v[sl] = rows_v[sl] * wv
            return carry
        lax.fori_loop(0, CHUNK, edge_body, 0, unroll=4)

        scatter_start(b, m)

    for t in range(4):
        step(t, t, t >= 1, True, True)

    def loop_body(jj, carry):
        for t in range(4):
            step(4 * jj + t, t, True, True, True)
        return carry
    lax.fori_loop(1, CPW // 4 - 1, loop_body, 0)
    for t in range(4):
        j = CPW - 4 + t
        step(j, t, True, j + 1 < CPW, j + 2 < CPW)

    scatter_wait(1, (CPW - 1) % 4)
    plsc.subcore_barrier()
    pltpu.sync_copy(acc_sh.at[pl.ds(s * RPT, RPT)],
                    out_hbm.at[c, pl.ds(s * RPT, RPT)])


_BN = 1000
_BC = RPT


def _combine_tc(p):
    def body(p_ref, o_ref):
        o_ref[...] = p_ref[0] + p_ref[1]
    return pl.pallas_call(
        body,
        grid=(N_PAD // _BC,),
        in_specs=[pl.BlockSpec((2, _BC, D), lambda i: (0, i, 0))],
        out_specs=pl.BlockSpec((_BC, D), lambda i: (i, 0)),
        out_shape=jax.ShapeDtypeStruct((N_PAD, D), jnp.float32),
    )(p)


def _linear_tc(h, W, b2):
    def body(h_ref, w_ref, b_ref, o_ref):
        acc = lax.dot_general(h_ref[...], w_ref[...],
                              (((1,), (1,)), ((), ())),
                              preferred_element_type=jnp.float32)
        o_ref[...] = acc + b_ref[...]
    return pl.pallas_call(
        body,
        grid=(N // _BN,),
        in_specs=[
            pl.BlockSpec((_BN, D), lambda i: (i, 0)),
            pl.BlockSpec((D, D), lambda i: (0, 0)),
            pl.BlockSpec((1, D), lambda i: (0, 0)),
        ],
        out_specs=pl.BlockSpec((_BN, D), lambda i: (i, 0)),
        out_shape=jax.ShapeDtypeStruct((N, D), jnp.float32),
    )(h, W, b2)


def kernel(x, edge_index, edge_weight, W, b, k):
    row = edge_index[0]
    col = edge_index[1]
    pad = E_PAD - E
    spread = (jnp.arange(pad, dtype=jnp.int32) * 37) % N
    col2 = jnp.concatenate([col, spread]).reshape(CHUNKS_PAD, 1, CHUNK)
    row2 = jnp.concatenate([row, spread]).reshape(CHUNKS_PAD, 1, CHUNK)
    idx = jnp.concatenate([col2, row2], axis=1)
    w2 = jnp.broadcast_to(
        jnp.pad(edge_weight, (0, pad)).reshape(CHUNKS_PAD, CHUNK, 1),
        (CHUNKS_PAD, CHUNK, LANES)).reshape(CHUNKS_PAD, LANES, CHUNK)
    zeros = jnp.zeros((N_PAD, D), jnp.float32)
    b2 = b.reshape(1, D)
    x_pad = jnp.pad(x, ((0, N_PAD - N), (0, 0)))

    def it_body(_, h):
        p = _spmm_sc(h, zeros, idx, w2)
        return _combine_tc(p)

    h = lax.fori_loop(0, k, it_body, x_pad)
    return _linear_tc(h[:N], W, b2)


# speedup vs baseline: 1.2050x; 1.0043x over previous
import functools

import jax
import jax.numpy as jnp
from jax import lax
from jax.experimental import pallas as pl
from jax.experimental.pallas import tpu as pltpu
from jax.experimental.pallas import tpu_sc as plsc

N = 10000
E = 320000
D = 128

NC = 2
NS = 16
NW = NC * NS
LANES = 16

CHUNK = 128
CPW = 80
CHUNKS_PAD = CPW * NW
E_PAD = CHUNKS_PAD * CHUNK
RPT = 8 * (-(-N // (8 * NS)))
N_PAD = RPT * NS

_mesh = plsc.VectorSubcoreMesh(
    core_axis_name="c", subcore_axis_name="s", num_cores=NC, num_subcores=NS)


@functools.partial(
    pl.kernel,
    out_type=jax.ShapeDtypeStruct((NC, N_PAD, D), jnp.float32),
    mesh=_mesh,
    scratch_types=[
        pltpu.VMEM((2, CHUNK, D), jnp.float32),
        pltpu.VMEM((4, 2, CHUNK), jnp.int32),
        pltpu.VMEM((4, LANES, CHUNK), jnp.float32),
        pltpu.VMEM_SHARED((N_PAD, D), jnp.float32),
        pltpu.SemaphoreType.DMA, pltpu.SemaphoreType.DMA,
        pltpu.SemaphoreType.DMA, pltpu.SemaphoreType.DMA,
        pltpu.SemaphoreType.DMA, pltpu.SemaphoreType.DMA,
        pltpu.SemaphoreType.DMA, pltpu.SemaphoreType.DMA,
    ],
)
def _spmm_sc(h_hbm, zeros_hbm, idx_hbm, w_hbm, out_hbm,
             rows_v, idx_v, w_v, acc_sh,
             sg0, sg1, si0, si1, si2, si3, ss0, ss1):
    c = lax.axis_index("c")
    s = lax.axis_index("s")
    sg = (sg0, sg1)
    si = (si0, si1, si2, si3)
    ss = (ss0, ss1)

    pltpu.sync_copy(zeros_hbm.at[pl.ds(s * RPT, RPT)],
                    acc_sh.at[pl.ds(s * RPT, RPT)])
    plsc.subcore_barrier()

    base = (c * NS + s) * CPW

    def meta_copy(m, j):
        pltpu.async_copy(idx_hbm.at[base + j], idx_v.at[m], si[m])
        pltpu.async_copy(w_hbm.at[base + j], w_v.at[m], si[m])

    def meta_wait(m, j):
        pltpu.make_async_copy(idx_hbm.at[base + j], idx_v.at[m],
                              si[m]).wait()
        pltpu.make_async_copy(w_hbm.at[base + j], w_v.at[m], si[m]).wait()

    def gather_start(b, m):
        pltpu.async_copy(h_hbm.at[idx_v.at[m, 0]], rows_v.at[b], sg[b])

    def gather_wait(b, m):
        pltpu.make_async_copy(h_hbm.at[idx_v.at[m, 0]], rows_v.at[b],
                              sg[b]).wait()

    meta_copy(0, 0)
    meta_wait(0, 0)
    meta_copy(1, 1)
    gather_start(0, 0)

    def scatter_start(b, m):
        pltpu.async_copy(rows_v.at[b], acc_sh.at[idx_v.at[m, 1]], ss[b],
                         add=True)

    def scatter_wait(b, m):
        pltpu.make_async_copy(rows_v.at[b], acc_sh.at[idx_v.at[m, 1]],
                              ss[b]).wait()

    def step(j, t, pf_retire, pf_gather, pf_meta):
        b = t % 2
        m = t % 4
        if pf_retire:
            scatter_wait(1 - b, (t + 3) % 4)
        if pf_gather:
            meta_wait((t + 1) % 4, j + 1)
            gather_start(1 - b, (t + 1) % 4)
        if pf_meta:
            meta_copy((t + 2) % 4, j + 2)

        gather_wait(b, m)

        def edge_body(i, carry):
            wv = w_v[m, i // 8, pl.ds((i % 8) * LANES, LANES)]
            for jj in range(D // LANES):
                sl = (b, i, pl.ds(jj * LANES, LANES))
                rows_v[sl] = rows_v[sl] * wv
            return carry
        lax.fori_loop(0, CHUNK, edge_body, 0, unroll=2)

        scatter_start(b, m)

    for t in range(4):
        step(t, t, t >= 1, True, True)

    def loop_body(jj, carry):
        for t in range(4):
            step(4 * jj + t, t, True, True, True)
        return carry
    lax.fori_loop(1, CPW // 4 - 1, loop_body, 0)
    for t in range(4):
        j = CPW - 4 + t
        step(j, t, True, j + 1 < CPW, j + 2 < CPW)

    scatter_wait(1, (CPW - 1) % 4)
    plsc.subcore_barrier()
    pltpu.sync_copy(acc_sh.at[pl.ds(s * RPT, RPT)],
                    out_hbm.at[c, pl.ds(s * RPT, RPT)])


_BN = 1000
_BC = RPT


def _combine_tc(p):
    def body(p_ref, o_ref):
        o_ref[...] = p_ref[0] + p_ref[1]
    return pl.pallas_call(
        body,
        grid=(N_PAD // _BC,),
        in_specs=[pl.BlockSpec((2, _BC, D), lambda i: (0, i, 0))],
        out_specs=pl.BlockSpec((_BC, D), lambda i: (i, 0)),
        out_shape=jax.ShapeDtypeStruct((N_PAD, D), jnp.float32),
    )(p)


def _linear_tc(h, W, b2):
    def body(h_ref, w_ref, b_ref, o_ref):
        acc = lax.dot_general(h_ref[...], w_ref[...],
                              (((1,), (1,)), ((), ())),
                              preferred_element_type=jnp.float32)
        o_ref[...] = acc + b_ref[...]
    return pl.pallas_call(
        body,
        grid=(N // _BN,),
        in_specs=[
            pl.BlockSpec((_BN, D), lambda i: (i, 0)),
            pl.BlockSpec((D, D), lambda i: (0, 0)),
            pl.BlockSpec((1, D), lambda i: (0, 0)),
        ],
        out_specs=pl.BlockSpec((_BN, D), lambda i: (i, 0)),
        out_shape=jax.ShapeDtypeStruct((N, D), jnp.float32),
    )(h, W, b2)


def kernel(x, edge_index, edge_weight, W, b, k):
    row = edge_index[0]
    col = edge_index[1]
    pad = E_PAD - E
    spread = (jnp.arange(pad, dtype=jnp.int32) * 37) % N
    col2 = jnp.concatenate([col, spread]).reshape(CHUNKS_PAD, 1, CHUNK)
    row2 = jnp.concatenate([row, spread]).reshape(CHUNKS_PAD, 1, CHUNK)
    idx = jnp.concatenate([col2, row2], axis=1)
    w2 = jnp.broadcast_to(
        jnp.pad(edge_weight, (0, pad)).reshape(CHUNKS_PAD, CHUNK, 1),
        (CHUNKS_PAD, CHUNK, LANES)).reshape(CHUNKS_PAD, LANES, CHUNK)
    zeros = jnp.zeros((N_PAD, D), jnp.float32)
    b2 = b.reshape(1, D)
    x_pad = jnp.pad(x, ((0, N_PAD - N), (0, 0)))

    def it_body(_, h):
        p = _spmm_sc(h, zeros, idx, w2)
        return _combine_tc(p)

    h = lax.fori_loop(0, k, it_body, x_pad)
    return _linear_tc(h[:N], W, b2)
